# baseline (speedup 1.0000x reference)
import functools

import jax
import jax.numpy as jnp
from jax import lax
from jax.experimental import pallas as pl
from jax.experimental.pallas import tpu as pltpu
from jax.experimental.pallas import tpu_sc as plsc

NC = 2
NS = 16
NW = NC * NS
LANES = 16

CHUNK = 40


def _make_sc_kernel(e: int, d: int):
    e_per_w = e // NW
    n_chunks = e_per_w // CHUNK
    n_pairs = n_chunks // 2
    mesh = plsc.VectorSubcoreMesh(core_axis_name="c", subcore_axis_name="s")

    @functools.partial(
        pl.kernel,
        mesh=mesh,
        out_type=jax.ShapeDtypeStruct((e, d), jnp.float32),
        scratch_types=[
            pltpu.VMEM((2, CHUNK), jnp.int32),
            pltpu.VMEM((2, CHUNK), jnp.int32),
            pltpu.VMEM((2, CHUNK, d), jnp.float32),
            pltpu.VMEM((2, CHUNK, d), jnp.float32),
            pltpu.SemaphoreType.DMA,
            pltpu.SemaphoreType.DMA,
            pltpu.SemaphoreType.DMA,
            pltpu.SemaphoreType.DMA,
            pltpu.SemaphoreType.DMA,
            pltpu.SemaphoreType.DMA,
            pltpu.SemaphoreType.DMA,
            pltpu.SemaphoreType.DMA,
        ],
    )
    def k(z_hbm, x_hbm, src_hbm, dst_hbm, out_hbm, idx_s, idx_d, zr, xr,
          gz0, gx0, gz1, gx1, o0, o1, i0, i1):
        gz = (gz0, gz1)
        gx = (gx0, gx1)
        osem = (o0, o1)
        isem = (i0, i1)
        wid = lax.axis_index("s") * NC + lax.axis_index("c")
        base = wid * e_per_w

        def start_idx(off, b):
            pltpu.async_copy(src_hbm.at[pl.ds(off, CHUNK)], idx_s.at[b], isem[b])
            pltpu.async_copy(dst_hbm.at[pl.ds(off, CHUNK)], idx_d.at[b], isem[b])

        def wait_idx(b):
            pltpu.make_async_copy(
                src_hbm.at[pl.ds(0, CHUNK)], idx_s.at[b], isem[b]).wait()
            pltpu.make_async_copy(
                dst_hbm.at[pl.ds(0, CHUNK)], idx_d.at[b], isem[b]).wait()

        def start_gather(b):
            pltpu.async_copy(z_hbm.at[idx_s.at[b]], zr.at[b], gz[b])
            pltpu.async_copy(x_hbm.at[idx_d.at[b]], xr.at[b], gx[b])

        def wait_gather(b):
            pltpu.make_async_copy(z_hbm.at[idx_s.at[b]], zr.at[b], gz[b]).wait()
            pltpu.make_async_copy(x_hbm.at[idx_d.at[b]], xr.at[b], gx[b]).wait()

        def wait_write(b):
            pltpu.make_async_copy(
                zr.at[b], out_hbm.at[pl.ds(0, CHUNK)], osem[b]).wait()

        def compute(b):
            def row_body(r, c2):
                for cc in range(d // LANES):
                    sl = pl.ds(cc * LANES, LANES)
                    zr[b, r, sl] = zr[b, r, sl] * xr[b, r, sl]
                return c2

            lax.fori_loop(0, CHUNK, row_body, 0, unroll=2)

        pltpu.sync_copy(src_hbm.at[pl.ds(base, CHUNK)], idx_s.at[0])
        pltpu.sync_copy(dst_hbm.at[pl.ds(base, CHUNK)], idx_d.at[0])
        start_gather(0)
        start_idx(base + CHUNK, 1)

        def chunk_body(p, b, off):
            nb = 1 - b

            if b == 0:
                @pl.when(p > 0)
                def _():
                    wait_write(nb)
            else:
                wait_write(nb)

            def advance():
                wait_idx(nb)
                start_gather(nb)

            if b == 0:
                advance()
            else:
                pl.when(p < n_pairs - 1)(advance)

            wait_gather(b)

            @pl.when(p < n_pairs - 1)
            def _():
                start_idx(off + 2 * CHUNK, b)

            compute(b)
            pltpu.async_copy(zr.at[b], out_hbm.at[pl.ds(off, CHUNK)], osem[b])

        def pair_body(p, carry):
            off0 = base + (2 * p) * CHUNK
            chunk_body(p, 0, off0)
            chunk_body(p, 1, off0 + CHUNK)
            return carry

        lax.fori_loop(0, n_pairs, pair_body, 0, unroll=False)
        wait_write(1)

    return k


def kernel(z, x, edge_index):
    e = edge_index.shape[1]
    d = z.shape[1]
    src = edge_index[0].astype(jnp.int32)
    dst = edge_index[1].astype(jnp.int32)
    return _make_sc_kernel(e, d)(z, x, src, dst)


# speedup vs baseline: 1.7463x; 1.7463x over previous
import functools

import jax
import jax.numpy as jnp
from jax import lax
from jax.experimental import pallas as pl
from jax.experimental.pallas import tpu as pltpu
from jax.experimental.pallas import tpu_sc as plsc

NC = 2
NS = 16
NW = NC * NS
LANES = 16

CHUNK = 40


def _make_sc_kernel(e: int, d: int):
    e_per_w = e // NW
    n_chunks = e_per_w // CHUNK
    n_pairs = n_chunks // 2
    mesh = plsc.VectorSubcoreMesh(core_axis_name="c", subcore_axis_name="s")

    @functools.partial(
        pl.kernel,
        mesh=mesh,
        out_type=jax.ShapeDtypeStruct((e, d), jnp.float32),
        scratch_types=[
            pltpu.VMEM((2, CHUNK), jnp.int32),
            pltpu.VMEM((2, CHUNK), jnp.int32),
            pltpu.VMEM((2, CHUNK, d), jnp.float32),
            pltpu.VMEM((2, CHUNK, d), jnp.float32),
            pltpu.SemaphoreType.DMA,
            pltpu.SemaphoreType.DMA,
            pltpu.SemaphoreType.DMA,
            pltpu.SemaphoreType.DMA,
            pltpu.SemaphoreType.DMA,
            pltpu.SemaphoreType.DMA,
            pltpu.SemaphoreType.DMA,
            pltpu.SemaphoreType.DMA,
        ],
    )
    def k(z_hbm, x_hbm, src_hbm, dst_hbm, out_hbm, idx_s, idx_d, zr, xr,
          gz0, gx0, gz1, gx1, o0, o1, i0, i1):
        gz = (gz0, gz1)
        gx = (gx0, gx1)
        osem = (o0, o1)
        isem = (i0, i1)
        wid = lax.axis_index("s") * NC + lax.axis_index("c")
        base = wid * e_per_w

        def start_idx(off, b):
            pltpu.async_copy(src_hbm.at[pl.ds(off, CHUNK)], idx_s.at[b], isem[b])
            pltpu.async_copy(dst_hbm.at[pl.ds(off, CHUNK)], idx_d.at[b], isem[b])

        def wait_idx(b):
            pltpu.make_async_copy(
                src_hbm.at[pl.ds(0, CHUNK)], idx_s.at[b], isem[b]).wait()
            pltpu.make_async_copy(
                dst_hbm.at[pl.ds(0, CHUNK)], idx_d.at[b], isem[b]).wait()

        def start_gather(b):
            pltpu.async_copy(z_hbm.at[idx_s.at[b]], zr.at[b], gz[b])
            pltpu.async_copy(x_hbm.at[idx_d.at[b]], xr.at[b], gx[b])

        def wait_gather(b):
            pltpu.make_async_copy(z_hbm.at[idx_s.at[b]], zr.at[b], gz[b]).wait()
            pltpu.make_async_copy(x_hbm.at[idx_d.at[b]], xr.at[b], gx[b]).wait()

        def wait_write(b):
            pltpu.make_async_copy(
                zr.at[b], out_hbm.at[pl.ds(0, CHUNK)], osem[b]).wait()

        def compute(b):
            def row_body(r, c2):
                for cc in range(d // LANES):
                    sl = pl.ds(cc * LANES, LANES)
                    zr[b, r, sl] = zr[b, r, sl] * xr[b, r, sl]
                return c2

            lax.fori_loop(0, CHUNK, row_body, 0, unroll=False)

        pltpu.sync_copy(src_hbm.at[pl.ds(base, CHUNK)], idx_s.at[0])
        pltpu.sync_copy(dst_hbm.at[pl.ds(base, CHUNK)], idx_d.at[0])
        start_gather(0)
        start_idx(base + CHUNK, 1)

        def chunk_body(p, b, off):
            nb = 1 - b

            if b == 0:
                @pl.when(p > 0)
                def _():
                    wait_write(nb)
            else:
                wait_write(nb)

            def advance():
                wait_idx(nb)
                start_gather(nb)

            if b == 0:
                advance()
            else:
                pl.when(p < n_pairs - 1)(advance)

            wait_gather(b)

            @pl.when(p < n_pairs - 1)
            def _():
                start_idx(off + 2 * CHUNK, b)

            compute(b)
            pltpu.async_copy(zr.at[b], out_hbm.at[pl.ds(off, CHUNK)], osem[b])

        def pair_body(p, carry):
            off0 = base + (2 * p) * CHUNK
            chunk_body(p, 0, off0)
            chunk_body(p, 1, off0 + CHUNK)
            return carry

        lax.fori_loop(0, n_pairs, pair_body, 0, unroll=False)
        wait_write(1)

    return k


def kernel(z, x, edge_index):
    e = edge_index.shape[1]
    d = z.shape[1]
    src = edge_index[0].astype(jnp.int32)
    dst = edge_index[1].astype(jnp.int32)
    return _make_sc_kernel(e, d)(z, x, src, dst)
